# baseline (speedup 1.0000x reference)
import functools

import jax
import jax.numpy as jnp
from jax import lax
from jax.experimental import pallas as pl
from jax.experimental.pallas import tpu as pltpu
from jax.experimental.pallas import tpu_sc as plsc

BATCH = 16384
NUM_FACTORS = 32
LANES = 16
NUM_WORKERS = 32
B_PER_W = BATCH // NUM_WORKERS
Q = 4
NITER = B_PER_W // (2 * Q)
IDX_PAD = B_PER_W + LANES


def _fire(tab_u, tab_i, lu, li, lane0, buf_u, buf_i, sem):
    for k in range(Q):
        su = lu[lane0 + k]
        si = li[lane0 + k]
        au = pl.multiple_of(su & -128, 128)
        ai = pl.multiple_of(si & -128, 128)
        for t in range(4):
            pltpu.async_copy(tab_u.at[pl.ds(8 * t, 8), pl.ds(au, 128)],
                             buf_u.at[k, pl.ds(8 * t, 8)], sem)
            pltpu.async_copy(tab_i.at[pl.ds(8 * t, 8), pl.ds(ai, 128)],
                             buf_i.at[k, pl.ds(8 * t, 8)], sem)


def _drain(tab_u, buf_u, buf_i, sem):
    for k in range(Q):
        for t in range(4):
            pltpu.make_async_copy(tab_u.at[pl.ds(0, 8), pl.ds(0, 128)],
                                  buf_u.at[k, pl.ds(8 * t, 8)], sem).wait()
            pltpu.make_async_copy(tab_u.at[pl.ds(0, 8), pl.ds(0, 128)],
                                  buf_i.at[k, pl.ds(8 * t, 8)], sem).wait()


def _compute(lu, li, lane0, acc_lane0, buf_u, buf_i, lane16, acc):
    for k in range(Q):
        cu = jnp.full((LANES,), lu[lane0 + k] & 127, jnp.int32)
        ci = jnp.full((LANES,), li[lane0 + k] & 127, jnp.int32)
        uu0 = plsc.load_gather(buf_u.at[k], [lane16, cu])
        uu1 = plsc.load_gather(buf_u.at[k], [lane16 + LANES, cu])
        vv0 = plsc.load_gather(buf_i.at[k], [lane16, ci])
        vv1 = plsc.load_gather(buf_i.at[k], [lane16 + LANES, ci])
        p = uu0 * vv0 + uu1 * vv1
        acc = jnp.where(lane16 == acc_lane0 + k, jnp.sum(p), acc)
    return acc


def _body(user_hbm, item_hbm, ut_hbm, it_hbm, out_hbm,
          idx_u, idx_i, bufa_u, bufa_i, bufb_u, bufb_i, out_v, sema, semb):
    wid = lax.axis_index("s") * 2 + lax.axis_index("c")
    base = wid * B_PER_W

    pltpu.sync_copy(user_hbm.at[pl.ds(base, B_PER_W)],
                    idx_u.at[pl.ds(0, B_PER_W)])
    pltpu.sync_copy(item_hbm.at[pl.ds(base, B_PER_W)],
                    idx_i.at[pl.ds(0, B_PER_W)])

    lane16 = lax.iota(jnp.int32, LANES)

    lu0 = idx_u[pl.ds(0, LANES)]
    li0 = idx_i[pl.ds(0, LANES)]
    _fire(ut_hbm, it_hbm, lu0, li0, 0, bufa_u, bufa_i, sema)

    def iter_body(j, acc):
        lu = idx_u[pl.ds(j * 8, LANES)]
        li = idx_i[pl.ds(j * 8, LANES)]
        acc_l0 = (j % 2) * 8

        _fire(ut_hbm, it_hbm, lu, li, 4, bufb_u, bufb_i, semb)
        _drain(ut_hbm, bufa_u, bufa_i, sema)
        acc = _compute(lu, li, 0, acc_l0, bufa_u, bufa_i, lane16, acc)

        @pl.when(j < NITER - 1)
        def _():
            _fire(ut_hbm, it_hbm, lu, li, 8, bufa_u, bufa_i, sema)

        _drain(ut_hbm, bufb_u, bufb_i, semb)
        acc = _compute(lu, li, 4, acc_l0 + 4, bufb_u, bufb_i, lane16, acc)

        @pl.when(j % 2 == 1)
        def _():
            out_v[pl.ds((j // 2) * LANES, LANES)] = acc

        return jnp.where(j % 2 == 1, jnp.zeros((LANES,), jnp.float32), acc)

    lax.fori_loop(0, NITER, iter_body, jnp.zeros((LANES,), jnp.float32))

    pltpu.sync_copy(out_v, out_hbm.at[pl.ds(base, B_PER_W)])


@jax.jit
def _mf_dot(user, item, user_factors, item_factors):
    ut = user_factors.T
    it = item_factors.T
    mesh = plsc.VectorSubcoreMesh(core_axis_name="c", subcore_axis_name="s")
    return pl.kernel(
        _body,
        out_type=jax.ShapeDtypeStruct((BATCH,), jnp.float32),
        mesh=mesh,
        compiler_params=pltpu.CompilerParams(needs_layout_passes=False),
        scratch_types=[
            pltpu.VMEM((IDX_PAD,), jnp.int32),
            pltpu.VMEM((IDX_PAD,), jnp.int32),
            pltpu.VMEM((Q, NUM_FACTORS, 128), jnp.float32),
            pltpu.VMEM((Q, NUM_FACTORS, 128), jnp.float32),
            pltpu.VMEM((Q, NUM_FACTORS, 128), jnp.float32),
            pltpu.VMEM((Q, NUM_FACTORS, 128), jnp.float32),
            pltpu.VMEM((B_PER_W,), jnp.float32),
            pltpu.SemaphoreType.DMA,
            pltpu.SemaphoreType.DMA,
        ],
    )(user, item, ut, it)


def kernel(user, item, user_factors, item_factors):
    return _mf_dot(user.astype(jnp.int32), item.astype(jnp.int32),
                   user_factors, item_factors)


# speedup vs baseline: 1.0210x; 1.0210x over previous
import jax
import jax.numpy as jnp
from jax import lax
from jax.experimental import pallas as pl
from jax.experimental.pallas import tpu as pltpu
from jax.experimental.pallas import tpu_sc as plsc

BATCH = 16384
NUM_FACTORS = 32
LANES = 16
NUM_WORKERS = 32
B_PER_W = BATCH // NUM_WORKERS
Q = 4
NITER = B_PER_W // (2 * Q)
IDX_PAD = B_PER_W + LANES


def _fire(tab_u, tab_i, lu, li, lane0, buf_u, buf_i, sem):
    for k in range(Q):
        su = lu[lane0 + k]
        si = li[lane0 + k]
        au = pl.multiple_of(su & -128, 128)
        ai = pl.multiple_of(si & -128, 128)
        pltpu.async_copy(tab_u.at[:, pl.ds(au, 128)], buf_u.at[k], sem)
        pltpu.async_copy(tab_i.at[:, pl.ds(ai, 128)], buf_i.at[k], sem)


def _drain(tab_u, buf_u, buf_i, sem):
    for k in range(Q):
        pltpu.make_async_copy(tab_u.at[:, pl.ds(0, 128)], buf_u.at[k], sem).wait()
        pltpu.make_async_copy(tab_u.at[:, pl.ds(0, 128)], buf_i.at[k], sem).wait()


def _compute(lu, li, lane0, acc_lane0, buf_u, buf_i, lane16, acc):
    for k in range(Q):
        cu = jnp.full((LANES,), lu[lane0 + k] & 127, jnp.int32)
        ci = jnp.full((LANES,), li[lane0 + k] & 127, jnp.int32)
        uu0 = plsc.load_gather(buf_u.at[k], [lane16, cu])
        uu1 = plsc.load_gather(buf_u.at[k], [lane16 + LANES, cu])
        vv0 = plsc.load_gather(buf_i.at[k], [lane16, ci])
        vv1 = plsc.load_gather(buf_i.at[k], [lane16 + LANES, ci])
        p = uu0 * vv0 + uu1 * vv1
        acc = jnp.where(lane16 == acc_lane0 + k, jnp.sum(p), acc)
    return acc


def _body(user_hbm, item_hbm, ut_hbm, it_hbm, out_hbm,
          idx_u, idx_i, bufa_u, bufa_i, bufb_u, bufb_i, out_v, sema, semb):
    wid = lax.axis_index("s") * 2 + lax.axis_index("c")
    base = wid * B_PER_W

    pltpu.sync_copy(user_hbm.at[pl.ds(base, B_PER_W)],
                    idx_u.at[pl.ds(0, B_PER_W)])
    pltpu.sync_copy(item_hbm.at[pl.ds(base, B_PER_W)],
                    idx_i.at[pl.ds(0, B_PER_W)])

    lane16 = lax.iota(jnp.int32, LANES)

    lu0 = idx_u[pl.ds(0, LANES)]
    li0 = idx_i[pl.ds(0, LANES)]
    _fire(ut_hbm, it_hbm, lu0, li0, 0, bufa_u, bufa_i, sema)

    def iter_body(j, acc):
        lu = idx_u[pl.ds(j * 8, LANES)]
        li = idx_i[pl.ds(j * 8, LANES)]
        acc_l0 = (j % 2) * 8

        _fire(ut_hbm, it_hbm, lu, li, 4, bufb_u, bufb_i, semb)
        _drain(ut_hbm, bufa_u, bufa_i, sema)
        acc = _compute(lu, li, 0, acc_l0, bufa_u, bufa_i, lane16, acc)

        @pl.when(j < NITER - 1)
        def _():
            _fire(ut_hbm, it_hbm, lu, li, 8, bufa_u, bufa_i, sema)

        _drain(ut_hbm, bufb_u, bufb_i, semb)
        acc = _compute(lu, li, 4, acc_l0 + 4, bufb_u, bufb_i, lane16, acc)

        @pl.when(j % 2 == 1)
        def _():
            out_v[pl.ds((j // 2) * LANES, LANES)] = acc

        return jnp.where(j % 2 == 1, jnp.zeros((LANES,), jnp.float32), acc)

    lax.fori_loop(0, NITER, iter_body, jnp.zeros((LANES,), jnp.float32))

    pltpu.sync_copy(out_v, out_hbm.at[pl.ds(base, B_PER_W)])


@jax.jit
def _mf_dot(user, item, user_factors, item_factors):
    ut = user_factors.T
    it = item_factors.T
    mesh = plsc.VectorSubcoreMesh(core_axis_name="c", subcore_axis_name="s")
    return pl.kernel(
        _body,
        out_type=jax.ShapeDtypeStruct((BATCH,), jnp.float32),
        mesh=mesh,
        compiler_params=pltpu.CompilerParams(needs_layout_passes=False),
        scratch_types=[
            pltpu.VMEM((IDX_PAD,), jnp.int32),
            pltpu.VMEM((IDX_PAD,), jnp.int32),
            pltpu.VMEM((Q, NUM_FACTORS, 128), jnp.float32),
            pltpu.VMEM((Q, NUM_FACTORS, 128), jnp.float32),
            pltpu.VMEM((Q, NUM_FACTORS, 128), jnp.float32),
            pltpu.VMEM((Q, NUM_FACTORS, 128), jnp.float32),
            pltpu.VMEM((B_PER_W,), jnp.float32),
            pltpu.SemaphoreType.DMA,
            pltpu.SemaphoreType.DMA,
        ],
    )(user, item, ut, it)


def kernel(user, item, user_factors, item_factors):
    return _mf_dot(user.astype(jnp.int32), item.astype(jnp.int32),
                   user_factors, item_factors)
